# baseline (speedup 1.0000x reference)
import functools

import jax
import jax.numpy as jnp
from jax import lax
from jax.experimental import pallas as pl
from jax.experimental.pallas import tpu as pltpu
from jax.experimental.pallas import tpu_sc as plsc

N = 10000
E = 320000
H = 4
C = 32
D = 128
ROW = 144
CH = 64
NCHUNK = E // CH
NW = 32
NP = 10112
RPT = NP // 16
WB = 64
TC_MAX = 157


def _lnk(x, g, b):
    m = x.mean(-1, keepdims=True)
    v = ((x - m) ** 2).mean(-1, keepdims=True)
    return (x - m) / jnp.sqrt(v + 1e-5) * g + b


def _blocksum(shape, jdim):
    j = lax.broadcasted_iota(jnp.int32, shape, jdim)
    h = lax.broadcasted_iota(jnp.int32, shape, 1 - jdim)
    return (j // C == h).astype(jnp.float32)


def _prep_body(src_r, dst_r, we_r, wk_r, bk_r, be_r, attf_r,
               srcrow_r, adst_r, dstvb_r):
    f32 = jnp.float32
    s = src_r[...]
    d = dst_r[...]
    attf = attf_r[...]
    wk = wk_r[...]
    g = _blocksum((D, H), 0)
    u1 = jnp.dot(wk[:D] * attf, g, preferred_element_type=f32)
    u2 = jnp.dot(wk[D:] * attf, g, preferred_element_type=f32)
    c0 = jnp.dot(bk_r[...] * attf, g, preferred_element_type=f32)
    we = we_r[...]
    wcat = jnp.concatenate([we[:D], u1, jnp.zeros((D, ROW - D - H), f32)], axis=1)
    wd16 = jnp.concatenate([u2, jnp.zeros((D, 16 - H), f32)], axis=1)
    c0p = jnp.concatenate([c0, jnp.zeros((1, 16 - H), f32)], axis=1)
    srcrow_r[...] = jnp.dot(s, wcat, preferred_element_type=f32)
    adst_r[...] = jnp.dot(d, wd16, preferred_element_type=f32) + c0p
    dstvb_r[...] = jnp.dot(d, we[D:], preferred_element_type=f32) + be_r[...]


def _prep(src, dst, W_enc, W_enc_k, b_enc_k, b_enc, attf):
    return pl.pallas_call(
        _prep_body,
        out_shape=[
            jax.ShapeDtypeStruct((N, ROW), jnp.float32),
            jax.ShapeDtypeStruct((N, 16), jnp.float32),
            jax.ShapeDtypeStruct((N, D), jnp.float32),
        ],
    )(src, dst, W_enc, W_enc_k, b_enc_k.reshape(1, D), b_enc.reshape(1, D), attf)


def _edge_pass(sidx, didx, srcrow, adst16, zrows):
    mesh = plsc.VectorSubcoreMesh(core_axis_name="c", subcore_axis_name="s",
                                  num_cores=2, num_subcores=16)

    @functools.partial(
        pl.kernel,
        out_type=jax.ShapeDtypeStruct((2, NP, ROW), jnp.float32),
        mesh=mesh,
        compiler_params=pltpu.CompilerParams(use_tc_tiling_on_sc=False,
                                             needs_layout_passes=False),
        scratch_types=[
            pltpu.VMEM((4, CH), jnp.int32),
            pltpu.VMEM((4, CH), jnp.int32),
            pltpu.VMEM((2, CH, ROW), jnp.float32),
            pltpu.VMEM((2, CH, 16), jnp.float32),
            pltpu.VMEM((2, CH, ROW), jnp.float32),
            pltpu.VMEM_SHARED((NP, ROW), jnp.float32),
            pltpu.SemaphoreType.DMA,
            pltpu.SemaphoreType.DMA,
            pltpu.SemaphoreType.DMA,
            pltpu.SemaphoreType.DMA,
            pltpu.SemaphoreType.DMA,
            pltpu.SemaphoreType.DMA,
            pltpu.SemaphoreType.DMA,
            pltpu.SemaphoreType.DMA,
            pltpu.SemaphoreType.DMA,
            pltpu.SemaphoreType.DMA,
        ],
    )
    def body(sidx_h, didx_h, srcrow_h, adst_h, zrows_h, out_h,
             svb4, dvb4, rows2, adv2, outr2, acc,
             semi0, semi1, semi2, semi3, semg0, semg1, sema0, sema1,
             sems0, sems1):
        semi = (semi0, semi1, semi2, semi3)
        semg = (semg0, semg1)
        sema = (sema0, sema1)
        sems = (sems0, sems1)
        cid = lax.axis_index("c")
        sid = lax.axis_index("s")
        wid = sid * 2 + cid
        lanes = lax.iota(jnp.int32, 16)
        ones16 = jnp.ones((16,), jnp.float32)
        lo = wid * NCHUNK // NW
        hi = (wid + 1) * NCHUNK // NW

        def issue_idx(ck, q):
            pltpu.async_copy(sidx_h.at[pl.ds(ck * CH, CH)], svb4.at[q], semi[q])
            pltpu.async_copy(didx_h.at[pl.ds(ck * CH, CH)], dvb4.at[q], semi[q])

        def wait_idx(ck, q):
            pltpu.make_async_copy(sidx_h.at[pl.ds(ck * CH, CH)], svb4.at[q], semi[q]).wait()
            pltpu.make_async_copy(didx_h.at[pl.ds(ck * CH, CH)], dvb4.at[q], semi[q]).wait()

        def issue_gathers(b, q):
            pltpu.async_copy(srcrow_h.at[svb4.at[q]], rows2.at[b], semg[b])
            pltpu.async_copy(adst_h.at[dvb4.at[q]], adv2.at[b], sema[b])

        def work(b, q):
            pltpu.make_async_copy(srcrow_h.at[svb4.at[q]], rows2.at[b], semg[b]).wait()
            pltpu.make_async_copy(adst_h.at[dvb4.at[q]], adv2.at[b], sema[b]).wait()

            def sub(g, c2):
                evec = g * 16 + lanes
                bvec = jnp.full((16,), b, jnp.int32)
                ws = []
                for h in range(H):
                    wcol = jnp.full((16,), D + h, jnp.int32)
                    a_s = plsc.load_gather(rows2, [bvec, evec, wcol])
                    a_d = plsc.load_gather(adv2, [bvec, evec, jnp.full((16,), h, jnp.int32)])
                    al = a_s + a_d
                    al = jnp.where(al >= 0.0, al, 0.2 * al)
                    w = jnp.exp(al)
                    ws.append(w)
                    plsc.store_scatter(outr2, [bvec, evec, wcol], w)
                plsc.store_scatter(outr2, [bvec, evec, jnp.full((16,), D + H, jnp.int32)], ones16)
                for e in range(16):
                    r = g * 16 + e
                    sel = jnp.full((16,), e, jnp.int32)
                    for h in range(H):
                        wb_ = ws[h].at[sel].get(mode='promise_in_bounds')
                        lohalf = rows2[b, r, pl.ds(h * C, 16)] * wb_
                        hihalf = rows2[b, r, pl.ds(h * C + 16, 16)] * wb_
                        outr2[b, r, pl.ds(h * C, 16)] = lohalf
                        outr2[b, r, pl.ds(h * C + 16, 16)] = hihalf
                return c2
            lax.fori_loop(0, CH // 16, sub, 0)
            pltpu.async_copy(outr2.at[b], acc.at[dvb4.at[q]], sems[b], add=True)

        issue_idx(lo, 0)

        @pl.when(lo + 1 < hi)
        def _():
            issue_idx(lo + 1, 1)
        pltpu.sync_copy(zrows_h, acc.at[pl.ds(sid * RPT, RPT)])
        pltpu.sync_copy(zrows_h.at[pl.ds(0, CH)], outr2.at[0])
        pltpu.sync_copy(zrows_h.at[pl.ds(0, CH)], outr2.at[1])
        plsc.subcore_barrier()
        wait_idx(lo, 0)
        issue_gathers(0, 0)

        def quad(j4, carry):
            for p in range(4):
                ck = lo + 4 * j4 + p
                b = p & 1

                @pl.when(jnp.logical_and(ck - 2 >= lo, ck - 2 < hi))
                def _():
                    pltpu.make_async_copy(outr2.at[b], acc.at[dvb4.at[(p + 2) % 4]],
                                          sems[b]).wait()

                @pl.when(ck + 2 < hi)
                def _():
                    issue_idx(ck + 2, (p + 2) % 4)

                @pl.when(ck + 1 < hi)
                def _():
                    wait_idx(ck + 1, (p + 1) % 4)
                    issue_gathers(1 - b, (p + 1) % 4)

                @pl.when(ck < hi)
                def _():
                    work(b, p)
            return carry
        lax.fori_loop(0, (TC_MAX + 3) // 4, quad, 0)
        plsc.subcore_barrier()

        def wb(i, carry):
            r0 = sid * RPT + i * WB
            pltpu.sync_copy(acc.at[pl.ds(r0, WB)], rows2.at[0])
            pltpu.sync_copy(rows2.at[0], out_h.at[cid, pl.ds(r0, WB)])
            return carry
        lax.fori_loop(0, RPT // WB, wb, 0)
        r0t = sid * RPT + (RPT // WB) * WB
        tl = RPT - (RPT // WB) * WB
        pltpu.sync_copy(acc.at[pl.ds(r0t, tl)], rows2.at[0, pl.ds(0, tl)])
        pltpu.sync_copy(rows2.at[0, pl.ds(0, tl)], out_h.at[cid, pl.ds(r0t, tl)])

    return body(sidx, didx, srcrow, adst16, zrows)


def _post_body(acc_r, dstvb_r, attf_r, l0g_r, l0b_r, l1g_r, l1b_r,
               w1_r, b1_r, lg_r, lb_r, w2_r, b2_r, out_r):
    f32 = jnp.float32
    a = acc_r[0, :N] + acc_r[1, :N]
    accv = a[:, :D]
    den = a[:, D:D + H]
    cnt = a[:, D + H:D + H + 1]
    den_b = jnp.dot(den, _blocksum((H, D), 1), preferred_element_type=f32)
    dv = dstvb_r[...]
    summed = (accv + den_b * dv) / (den_b + 1e-16)
    out0 = summed / jnp.maximum(cnt, 1.0) + attf_r[...]
    out0 = _lnk(out0, l0g_r[...], l0b_r[...])
    h1 = jnp.dot(out0, w1_r[...], preferred_element_type=f32) + b1_r[...]
    h1 = _lnk(h1, lg_r[...], lb_r[...])
    h1 = jnp.maximum(h1, 0.0)
    h2 = jnp.dot(h1, w2_r[...], preferred_element_type=f32) + b2_r[...]
    out_r[...] = _lnk(out0 + jnp.maximum(h2, 0.0), l1g_r[...], l1b_r[...])


def _post(acc2, dstvb, attf, l0g, l0b, l1g, l1b, w1, b1, lg, lb, w2, b2):
    return pl.pallas_call(
        _post_body,
        out_shape=jax.ShapeDtypeStruct((N, D), jnp.float32),
    )(acc2, dstvb, attf, l0g.reshape(1, D), l0b.reshape(1, D),
      l1g.reshape(1, D), l1b.reshape(1, D), w1, b1.reshape(1, D),
      lg.reshape(1, D), lb.reshape(1, D), w2, b2.reshape(1, D))


def kernel(src, dst, edge_index, num_nodes, W_enc, b_enc, W_enc_k, b_enc_k,
           att_r, ln0_g, ln0_b, ln1_g, ln1_b, mW1, mb1, mlng, mlnb, mW2, mb2):
    attf = att_r.reshape(1, D)
    srcrow, adst16, dstvb = _prep(src, dst, W_enc, W_enc_k, b_enc_k, b_enc, attf)

    sidx = edge_index[0]
    didx = edge_index[1]
    zrows = jnp.zeros((RPT, ROW), jnp.float32)
    acc2 = _edge_pass(sidx, didx, srcrow, adst16, zrows)

    out = _post(acc2, dstvb, attf, ln0_g, ln0_b, ln1_g, ln1_b,
                mW1, mb1, mlng, mlnb, mW2, mb2)
    return out


# speedup vs baseline: 1.0167x; 1.0167x over previous
import functools

import jax
import jax.numpy as jnp
from jax import lax
from jax.experimental import pallas as pl
from jax.experimental.pallas import tpu as pltpu
from jax.experimental.pallas import tpu_sc as plsc

N = 10000
E = 320000
H = 4
C = 32
D = 128
ROW = 144
CH = 64
NCHUNK = E // CH
NW = 32
NP = 10112
RPT = NP // 16
WB = 64
TC_MAX = 157


def _lnk(x, g, b):
    m = x.mean(-1, keepdims=True)
    v = ((x - m) ** 2).mean(-1, keepdims=True)
    return (x - m) / jnp.sqrt(v + 1e-5) * g + b


def _blocksum(shape, jdim):
    j = lax.broadcasted_iota(jnp.int32, shape, jdim)
    h = lax.broadcasted_iota(jnp.int32, shape, 1 - jdim)
    return (j // C == h).astype(jnp.float32)


def _prep_body(src_r, dst_r, we_r, wk_r, bk_r, be_r, attf_r,
               srcrow_r, adst_r, dstvb_r):
    f32 = jnp.float32
    s = src_r[...]
    d = dst_r[...]
    attf = attf_r[...]
    wk = wk_r[...]
    g = _blocksum((D, H), 0)
    u1 = jnp.dot(wk[:D] * attf, g, preferred_element_type=f32)
    u2 = jnp.dot(wk[D:] * attf, g, preferred_element_type=f32)
    c0 = jnp.dot(bk_r[...] * attf, g, preferred_element_type=f32)
    we = we_r[...]
    wcat = jnp.concatenate([we[:D], u1, jnp.zeros((D, ROW - D - H), f32)], axis=1)
    wd16 = jnp.concatenate([u2, jnp.zeros((D, 16 - H), f32)], axis=1)
    c0p = jnp.concatenate([c0, jnp.zeros((1, 16 - H), f32)], axis=1)
    srcrow_r[...] = jnp.dot(s, wcat, preferred_element_type=f32)
    adst_r[...] = jnp.dot(d, wd16, preferred_element_type=f32) + c0p
    dstvb_r[...] = jnp.dot(d, we[D:], preferred_element_type=f32) + be_r[...]


def _prep(src, dst, W_enc, W_enc_k, b_enc_k, b_enc, attf):
    return pl.pallas_call(
        _prep_body,
        out_shape=[
            jax.ShapeDtypeStruct((N, ROW), jnp.float32),
            jax.ShapeDtypeStruct((N, 16), jnp.float32),
            jax.ShapeDtypeStruct((N, D), jnp.float32),
        ],
    )(src, dst, W_enc, W_enc_k, b_enc_k.reshape(1, D), b_enc.reshape(1, D), attf)


def _edge_pass(sidx, didx, srcrow, adst16, zrows):
    mesh = plsc.VectorSubcoreMesh(core_axis_name="c", subcore_axis_name="s",
                                  num_cores=2, num_subcores=16)

    @functools.partial(
        pl.kernel,
        out_type=jax.ShapeDtypeStruct((2, NP, ROW), jnp.float32),
        mesh=mesh,
        compiler_params=pltpu.CompilerParams(use_tc_tiling_on_sc=False,
                                             needs_layout_passes=False),
        scratch_types=[
            pltpu.VMEM((4, CH), jnp.int32),
            pltpu.VMEM((4, CH), jnp.int32),
            pltpu.VMEM((2, CH, ROW), jnp.float32),
            pltpu.VMEM((2, CH, 16), jnp.float32),
            pltpu.VMEM((2, CH, ROW), jnp.float32),
            pltpu.VMEM_SHARED((NP, ROW), jnp.float32),
            pltpu.SemaphoreType.DMA,
            pltpu.SemaphoreType.DMA,
            pltpu.SemaphoreType.DMA,
            pltpu.SemaphoreType.DMA,
            pltpu.SemaphoreType.DMA,
            pltpu.SemaphoreType.DMA,
            pltpu.SemaphoreType.DMA,
            pltpu.SemaphoreType.DMA,
            pltpu.SemaphoreType.DMA,
            pltpu.SemaphoreType.DMA,
        ],
    )
    def body(sidx_h, didx_h, srcrow_h, adst_h, zrows_h, out_h,
             svb4, dvb4, rows2, adv2, outr2, acc,
             semi0, semi1, semi2, semi3, semg0, semg1, sema0, sema1,
             sems0, sems1):
        semi = (semi0, semi1, semi2, semi3)
        semg = (semg0, semg1)
        sema = (sema0, sema1)
        sems = (sems0, sems1)
        cid = lax.axis_index("c")
        sid = lax.axis_index("s")
        wid = sid * 2 + cid
        pltpu.sync_copy(zrows_h, acc.at[pl.ds(sid * RPT, RPT)])

        def zr(r, carry):
            def zc(k, c2):
                outr2[0, r, pl.ds(k * 16, 16)] = jnp.zeros((16,), jnp.float32)
                outr2[1, r, pl.ds(k * 16, 16)] = jnp.zeros((16,), jnp.float32)
                return c2
            return lax.fori_loop(0, ROW // 16, zc, carry)
        lax.fori_loop(0, CH, zr, 0)
        plsc.subcore_barrier()

        lanes = lax.iota(jnp.int32, 16)
        ones16 = jnp.ones((16,), jnp.float32)
        lo = wid * NCHUNK // NW
        hi = (wid + 1) * NCHUNK // NW

        def issue_idx(ck, q):
            pltpu.async_copy(sidx_h.at[pl.ds(ck * CH, CH)], svb4.at[q], semi[q])
            pltpu.async_copy(didx_h.at[pl.ds(ck * CH, CH)], dvb4.at[q], semi[q])

        def wait_idx(ck, q):
            pltpu.make_async_copy(sidx_h.at[pl.ds(ck * CH, CH)], svb4.at[q], semi[q]).wait()
            pltpu.make_async_copy(didx_h.at[pl.ds(ck * CH, CH)], dvb4.at[q], semi[q]).wait()

        def issue_gathers(b, q):
            pltpu.async_copy(srcrow_h.at[svb4.at[q]], rows2.at[b], semg[b])
            pltpu.async_copy(adst_h.at[dvb4.at[q]], adv2.at[b], sema[b])

        def work(b, q):
            pltpu.make_async_copy(srcrow_h.at[svb4.at[q]], rows2.at[b], semg[b]).wait()
            pltpu.make_async_copy(adst_h.at[dvb4.at[q]], adv2.at[b], sema[b]).wait()

            def sub(g, c2):
                evec = g * 16 + lanes
                bvec = jnp.full((16,), b, jnp.int32)
                ws = []
                for h in range(H):
                    wcol = jnp.full((16,), D + h, jnp.int32)
                    a_s = plsc.load_gather(rows2, [bvec, evec, wcol])
                    a_d = plsc.load_gather(adv2, [bvec, evec, jnp.full((16,), h, jnp.int32)])
                    al = a_s + a_d
                    al = jnp.where(al >= 0.0, al, 0.2 * al)
                    w = jnp.exp(al)
                    ws.append(w)
                    plsc.store_scatter(outr2, [bvec, evec, wcol], w)
                plsc.store_scatter(outr2, [bvec, evec, jnp.full((16,), D + H, jnp.int32)], ones16)
                for e in range(16):
                    r = g * 16 + e
                    sel = jnp.full((16,), e, jnp.int32)
                    for h in range(H):
                        wb_ = ws[h].at[sel].get(mode='promise_in_bounds')
                        lohalf = rows2[b, r, pl.ds(h * C, 16)] * wb_
                        hihalf = rows2[b, r, pl.ds(h * C + 16, 16)] * wb_
                        outr2[b, r, pl.ds(h * C, 16)] = lohalf
                        outr2[b, r, pl.ds(h * C + 16, 16)] = hihalf
                return c2
            lax.fori_loop(0, CH // 16, sub, 0)
            pltpu.async_copy(outr2.at[b], acc.at[dvb4.at[q]], sems[b], add=True)

        issue_idx(lo, 0)

        @pl.when(lo + 1 < hi)
        def _():
            issue_idx(lo + 1, 1)
        wait_idx(lo, 0)
        issue_gathers(0, 0)

        def quad(j4, carry):
            for p in range(4):
                ck = lo + 4 * j4 + p
                b = p & 1

                @pl.when(jnp.logical_and(ck - 2 >= lo, ck - 2 < hi))
                def _():
                    pltpu.make_async_copy(outr2.at[b], acc.at[dvb4.at[(p + 2) % 4]],
                                          sems[b]).wait()

                @pl.when(ck + 2 < hi)
                def _():
                    issue_idx(ck + 2, (p + 2) % 4)

                @pl.when(ck + 1 < hi)
                def _():
                    wait_idx(ck + 1, (p + 1) % 4)
                    issue_gathers(1 - b, (p + 1) % 4)

                @pl.when(ck < hi)
                def _():
                    work(b, p)
            return carry
        lax.fori_loop(0, (TC_MAX + 3) // 4, quad, 0)
        plsc.subcore_barrier()

        def wb(i, carry):
            r0 = sid * RPT + i * WB
            pltpu.sync_copy(acc.at[pl.ds(r0, WB)], rows2.at[0])
            pltpu.sync_copy(rows2.at[0], out_h.at[cid, pl.ds(r0, WB)])
            return carry
        lax.fori_loop(0, RPT // WB, wb, 0)
        r0t = sid * RPT + (RPT // WB) * WB
        tl = RPT - (RPT // WB) * WB
        pltpu.sync_copy(acc.at[pl.ds(r0t, tl)], rows2.at[0, pl.ds(0, tl)])
        pltpu.sync_copy(rows2.at[0, pl.ds(0, tl)], out_h.at[cid, pl.ds(r0t, tl)])

    return body(sidx, didx, srcrow, adst16, zrows)


def _post_body(acc_r, dstvb_r, attf_r, l0g_r, l0b_r, l1g_r, l1b_r,
               w1_r, b1_r, lg_r, lb_r, w2_r, b2_r, out_r):
    f32 = jnp.float32
    a = acc_r[0, :N] + acc_r[1, :N]
    accv = a[:, :D]
    den = a[:, D:D + H]
    cnt = a[:, D + H:D + H + 1]
    den_b = jnp.dot(den, _blocksum((H, D), 1), preferred_element_type=f32)
    dv = dstvb_r[...]
    summed = (accv + den_b * dv) / (den_b + 1e-16)
    out0 = summed / jnp.maximum(cnt, 1.0) + attf_r[...]
    out0 = _lnk(out0, l0g_r[...], l0b_r[...])
    h1 = jnp.dot(out0, w1_r[...], preferred_element_type=f32) + b1_r[...]
    h1 = _lnk(h1, lg_r[...], lb_r[...])
    h1 = jnp.maximum(h1, 0.0)
    h2 = jnp.dot(h1, w2_r[...], preferred_element_type=f32) + b2_r[...]
    out_r[...] = _lnk(out0 + jnp.maximum(h2, 0.0), l1g_r[...], l1b_r[...])


def _post(acc2, dstvb, attf, l0g, l0b, l1g, l1b, w1, b1, lg, lb, w2, b2):
    return pl.pallas_call(
        _post_body,
        out_shape=jax.ShapeDtypeStruct((N, D), jnp.float32),
    )(acc2, dstvb, attf, l0g.reshape(1, D), l0b.reshape(1, D),
      l1g.reshape(1, D), l1b.reshape(1, D), w1, b1.reshape(1, D),
      lg.reshape(1, D), lb.reshape(1, D), w2, b2.reshape(1, D))


def kernel(src, dst, edge_index, num_nodes, W_enc, b_enc, W_enc_k, b_enc_k,
           att_r, ln0_g, ln0_b, ln1_g, ln1_b, mW1, mb1, mlng, mlnb, mW2, mb2):
    attf = att_r.reshape(1, D)
    srcrow, adst16, dstvb = _prep(src, dst, W_enc, W_enc_k, b_enc_k, b_enc, attf)

    sidx = edge_index[0]
    didx = edge_index[1]
    zrows = jnp.zeros((RPT, ROW), jnp.float32)
    acc2 = _edge_pass(sidx, didx, srcrow, adst16, zrows)

    out = _post(acc2, dstvb, attf, ln0_g, ln0_b, ln1_g, ln1_b,
                mW1, mb1, mlng, mlnb, mW2, mb2)
    return out
